# baseline (speedup 1.0000x reference)
import functools

import jax
import jax.numpy as jnp
from jax import lax
from jax.experimental import pallas as pl
from jax.experimental.pallas import tpu as pltpu
from jax.experimental.pallas import tpu_sc as plsc

BATCH = 16384
EMBED_DIM = 32
NUM_WORKERS = 32
PER_W = BATCH // NUM_WORKERS
GATHER_CHUNK = 128
N_CHUNK = PER_W // GATHER_CHUNK
N_BLK = PER_W // 16


def _sc_body(uids_hbm, iids_hbm, ut_hbm, it_hbm, w_hbm, out_hbm,
             uidx_v, iidx_v, u_rows, i_rows, w_v, acc_v, sem):
    wid = lax.axis_index("s") * 2 + lax.axis_index("c")
    base = wid * PER_W

    pltpu.sync_copy(w_hbm, w_v)
    pltpu.sync_copy(uids_hbm.at[wid], uidx_v)
    pltpu.sync_copy(iids_hbm.at[wid], iidx_v)

    copies = []
    for j in range(N_CHUNK):
        dst = pl.ds(j * GATHER_CHUNK, GATHER_CHUNK)
        copies.append(pltpu.async_copy(ut_hbm.at[uidx_v.at[j]], u_rows.at[dst], sem))
        copies.append(pltpu.async_copy(it_hbm.at[iidx_v.at[j]], i_rows.at[dst], sem))
    for c in copies:
        c.wait()

    iot = lax.iota(jnp.int32, 16)

    b_bc = w_v[2 * EMBED_DIM, :]

    def init(blk, carry):
        acc_v[pl.ds(blk * 16, 16)] = b_bc
        return carry

    lax.fori_loop(0, N_BLK, init, 0)

    for d in range(EMBED_DIM):
        wu = w_v[d, :]
        wi = w_v[EMBED_DIM + d, :]
        cidx = jnp.full((16,), d, jnp.int32)

        def dot_body(blk, carry, wu=wu, wi=wi, cidx=cidx):
            ridx = iot + blk * 16
            v = plsc.load_gather(u_rows, [ridx, cidx]) * wu
            v = v + plsc.load_gather(i_rows, [ridx, cidx]) * wi
            plsc.addupdate(acc_v.at[pl.ds(blk * 16, 16)], v)
            return carry

        lax.fori_loop(0, N_BLK, dot_body, 0)

    def sig(blk, carry):
        s = pl.ds(blk * 16, 16)
        x = acc_v[s]
        acc_v[s] = 1.0 / (1.0 + jnp.exp(-x))
        return carry

    lax.fori_loop(0, N_BLK, sig, 0)
    pltpu.sync_copy(acc_v, out_hbm.at[pl.ds(base, PER_W)])


@jax.jit
def _run(uids3, iids3, user_table, item_table, wpad):
    mesh = plsc.VectorSubcoreMesh(core_axis_name="c", subcore_axis_name="s")
    k = functools.partial(
        pl.kernel,
        mesh=mesh,
        compiler_params=pltpu.CompilerParams(
            needs_layout_passes=False, use_tc_tiling_on_sc=False
        ),
        out_type=jax.ShapeDtypeStruct((BATCH,), jnp.float32),
        scratch_types=[
            pltpu.VMEM((N_CHUNK, GATHER_CHUNK), jnp.int32),
            pltpu.VMEM((N_CHUNK, GATHER_CHUNK), jnp.int32),
            pltpu.VMEM((PER_W, EMBED_DIM), jnp.float32),
            pltpu.VMEM((PER_W, EMBED_DIM), jnp.float32),
            pltpu.VMEM((72, 16), jnp.float32),
            pltpu.VMEM((PER_W,), jnp.float32),
            pltpu.SemaphoreType.DMA,
        ],
    )(_sc_body)
    return k(uids3, iids3, user_table, item_table, wpad)


def kernel(user_ids, item_ids, user_table, item_table, fc_w, fc_b):
    uids3 = user_ids.astype(jnp.int32).reshape(NUM_WORKERS, N_CHUNK, GATHER_CHUNK)
    iids3 = item_ids.astype(jnp.int32).reshape(NUM_WORKERS, N_CHUNK, GATHER_CHUNK)
    wflat = jnp.zeros((72,), jnp.float32)
    wflat = wflat.at[: 2 * EMBED_DIM].set(fc_w.reshape(-1))
    wflat = wflat.at[2 * EMBED_DIM].set(fc_b.reshape(())[()])
    wpad = jnp.tile(wflat.reshape(72, 1), (1, 16))
    out = _run(uids3, iids3, user_table, item_table, wpad)
    return out.reshape(BATCH, 1)
